# Initial kernel scaffold; baseline (speedup 1.0000x reference)
import jax
import jax.numpy as jnp
from jax.experimental import pallas as pl


def kernel(g, h, r, norm, W):
    raise NotImplementedError("write your pallas kernel here")


# speedup vs baseline: 1.2896x; 1.2896x over previous
import functools

import jax
import jax.numpy as jnp
from jax import lax
from jax.experimental import pallas as pl
from jax.experimental.pallas import tpu as pltpu
from jax.experimental.pallas import tpu_sc as plsc

NUM_NODES = 100000
H_DIM = 128
CHUNK = 128
NCHUNK = (NUM_NODES + CHUNK - 1) // CHUNK
PAD = NCHUNK * CHUNK
TAIL = NUM_NODES - (NCHUNK - 1) * CHUNK
NW = 32
CH_PER_W = (NCHUNK + NW - 1) // NW


def _gather_body(idx_hbm, table_hbm, out_hbm, idx_v, rows_v, sem):
    wid = lax.axis_index("s") * 2 + lax.axis_index("c")

    def body(i, carry):
        c = wid + i * NW

        @pl.when(c < NCHUNK - 1)
        def _():
            pltpu.sync_copy(idx_hbm.at[c], idx_v)
            pltpu.async_copy(table_hbm.at[idx_v], rows_v, sem).wait()
            pltpu.sync_copy(rows_v, out_hbm.at[pl.ds(c * CHUNK, CHUNK)])

        @pl.when(c == NCHUNK - 1)
        def _():
            pltpu.sync_copy(idx_hbm.at[c], idx_v)
            pltpu.async_copy(table_hbm.at[idx_v], rows_v, sem).wait()
            pltpu.sync_copy(rows_v.at[pl.ds(0, TAIL)],
                            out_hbm.at[pl.ds((NCHUNK - 1) * CHUNK, TAIL)])

        return carry

    lax.fori_loop(0, CH_PER_W, body, 0)


_mesh = plsc.VectorSubcoreMesh(core_axis_name="c", subcore_axis_name="s")

_gather = functools.partial(
    pl.kernel,
    mesh=_mesh,
    out_type=jax.ShapeDtypeStruct((NUM_NODES, H_DIM), jnp.float32),
    scratch_types=[
        pltpu.VMEM((CHUNK,), jnp.int32),
        pltpu.VMEM((CHUNK, H_DIM), jnp.float32),
        pltpu.SemaphoreType.DMA,
    ],
)(_gather_body)


@jax.jit
def kernel(g, h, r, norm, W):
    idx = h.reshape(-1).astype(jnp.int32)
    idx = jnp.concatenate(
        [idx, jnp.zeros((PAD - NUM_NODES,), jnp.int32)]).reshape(NCHUNK, CHUNK)
    return _gather(idx, W)


# speedup vs baseline: 1.8773x; 1.4558x over previous
import functools

import jax
import jax.numpy as jnp
from jax import lax
from jax.experimental import pallas as pl
from jax.experimental.pallas import tpu as pltpu
from jax.experimental.pallas import tpu_sc as plsc

NUM_NODES = 100000
H_DIM = 128
CHUNK = 128
NCHUNK = (NUM_NODES + CHUNK - 1) // CHUNK
TAIL = NUM_NODES - (NCHUNK - 1) * CHUNK
NW = 32
SLOTS = 25
NPADCHUNK = NW * SLOTS
PAD = NPADCHUNK * CHUNK
NBUF = 6
NITER = (SLOTS + NBUF - 1) // NBUF
BIG = NCHUNK // NW + 1
NBIGW = NCHUNK - NW * (BIG - 1)


def _gather_body(idx_hbm, table_hbm, out_hbm, idx_v, rows_v, *sems):
    gsem = sems[:NBUF]
    wsem = sems[NBUF:]
    wid = lax.axis_index("s") * 2 + lax.axis_index("c")
    start = wid * (BIG - 1) + jnp.minimum(wid, NBIGW)
    n_w = jnp.where(wid < NBIGW, BIG, BIG - 1)

    pltpu.sync_copy(idx_hbm.at[pl.ds(start * CHUNK, SLOTS * CHUNK)], idx_v)

    def fire(s, b):
        @pl.when(s < n_w)
        def _():
            pltpu.make_async_copy(
                table_hbm.at[idx_v.at[pl.ds(s * CHUNK, CHUNK)]],
                rows_v.at[b], gsem[b]).start()

    for b in range(NBUF):
        fire(b, b)

    def body(j, carry):
        for b in range(NBUF):
            s = j * NBUF + b
            c = start + s
            valid = s < n_w

            @pl.when(jnp.logical_and(valid, c < NCHUNK - 1))
            def _(b=b, s=s, c=c):
                pltpu.make_async_copy(
                    table_hbm.at[idx_v.at[pl.ds(s * CHUNK, CHUNK)]],
                    rows_v.at[b], gsem[b]).wait()
                pltpu.make_async_copy(
                    rows_v.at[b], out_hbm.at[pl.ds(c * CHUNK, CHUNK)],
                    wsem[b]).start()

            @pl.when(jnp.logical_and(valid, c == NCHUNK - 1))
            def _(b=b, s=s, c=c):
                pltpu.make_async_copy(
                    table_hbm.at[idx_v.at[pl.ds(s * CHUNK, CHUNK)]],
                    rows_v.at[b], gsem[b]).wait()
                pltpu.make_async_copy(
                    rows_v.at[b].at[pl.ds(0, TAIL)],
                    out_hbm.at[pl.ds((NCHUNK - 1) * CHUNK, TAIL)],
                    wsem[b]).start()

        for b in range(NBUF):
            s = j * NBUF + b
            c = start + s
            valid = s < n_w

            @pl.when(jnp.logical_and(valid, c < NCHUNK - 1))
            def _(b=b, s=s, c=c):
                pltpu.make_async_copy(
                    rows_v.at[b], out_hbm.at[pl.ds(c * CHUNK, CHUNK)],
                    wsem[b]).wait()

            @pl.when(jnp.logical_and(valid, c == NCHUNK - 1))
            def _(b=b, s=s, c=c):
                pltpu.make_async_copy(
                    rows_v.at[b].at[pl.ds(0, TAIL)],
                    out_hbm.at[pl.ds((NCHUNK - 1) * CHUNK, TAIL)],
                    wsem[b]).wait()

            fire(s + NBUF, b)
        return carry

    lax.fori_loop(0, NITER, body, 0)


_mesh = plsc.VectorSubcoreMesh(core_axis_name="c", subcore_axis_name="s")

_gather = functools.partial(
    pl.kernel,
    mesh=_mesh,
    out_type=jax.ShapeDtypeStruct((NUM_NODES, H_DIM), jnp.float32),
    scratch_types=[
        pltpu.VMEM((SLOTS * CHUNK,), jnp.int32),
        pltpu.VMEM((NBUF, CHUNK, H_DIM), jnp.float32),
    ] + [pltpu.SemaphoreType.DMA] * (2 * NBUF),
)(_gather_body)


@jax.jit
def kernel(g, h, r, norm, W):
    idx = h.reshape(-1).astype(jnp.int32)
    idx = jnp.concatenate(
        [idx, jnp.zeros((PAD - NUM_NODES,), jnp.int32)])
    return _gather(idx, W)


# speedup vs baseline: 2.1127x; 1.1254x over previous
import functools

import jax
import jax.numpy as jnp
from jax import lax
from jax.experimental import pallas as pl
from jax.experimental.pallas import tpu as pltpu
from jax.experimental.pallas import tpu_sc as plsc

NUM_NODES = 100000
H_DIM = 128
CHUNK = 128
NCHUNK = (NUM_NODES + CHUNK - 1) // CHUNK
TAIL = NUM_NODES - (NCHUNK - 1) * CHUNK
NW = 32
SLOTS = 25
BIG = NCHUNK // NW + 1
NBIGW = NCHUNK - NW * (BIG - 1)
LAST_START = (NW - 1) * (BIG - 1) + NBIGW
LASTN = NUM_NODES - LAST_START * CHUNK
NBUF = 7
WD = 3


def _gather_body(idx_hbm, table_hbm, out_hbm, idx_v, rows_v, gsems, wsems):
    wid = lax.axis_index("s") * 2 + lax.axis_index("c")
    start = wid * (BIG - 1) + jnp.minimum(wid, NBIGW)
    n_w = jnp.where(wid < NBIGW, BIG, BIG - 1)
    last_w = wid == NW - 1

    @pl.when(jnp.logical_not(last_w))
    def _():
        pltpu.sync_copy(
            idx_hbm.at[pl.ds(start * CHUNK, SLOTS * CHUNK)], idx_v)

    @pl.when(last_w)
    def _():
        pltpu.sync_copy(idx_hbm.at[pl.ds(LAST_START * CHUNK, LASTN)],
                        idx_v.at[pl.ds(0, LASTN)])

    def gdesc(s, b, n):
        return pltpu.make_async_copy(
            table_hbm.at[idx_v.at[pl.ds(s * CHUNK, n)]],
            rows_v.at[b].at[pl.ds(0, n)], gsems.at[b])

    def wdesc(s, b, c, n):
        return pltpu.make_async_copy(
            rows_v.at[b].at[pl.ds(0, n)],
            out_hbm.at[pl.ds(c * CHUNK, n)], wsems.at[b])

    def fire(s):
        b = lax.rem(s, NBUF)
        c = start + s

        @pl.when(jnp.logical_and(s < n_w, c < NCHUNK - 1))
        def _():
            gdesc(s, b, CHUNK).start()

        @pl.when(jnp.logical_and(s < n_w, c == NCHUNK - 1))
        def _():
            gdesc(s, b, TAIL).start()

    def wait_gather(s):
        b = lax.rem(s, NBUF)
        c = start + s

        @pl.when(c < NCHUNK - 1)
        def _():
            gdesc(s, b, CHUNK).wait()

        @pl.when(c == NCHUNK - 1)
        def _():
            gdesc(s, b, TAIL).wait()

    def start_write(s):
        b = lax.rem(s, NBUF)
        c = start + s

        @pl.when(c < NCHUNK - 1)
        def _():
            wdesc(s, b, c, CHUNK).start()

        @pl.when(c == NCHUNK - 1)
        def _():
            wdesc(s, b, c, TAIL).start()

    def wait_write(s):
        b = lax.rem(s, NBUF)
        c = start + s

        @pl.when(c < NCHUNK - 1)
        def _():
            wdesc(s, b, c, CHUNK).wait()

        @pl.when(c == NCHUNK - 1)
        def _():
            wdesc(s, b, c, TAIL).wait()

    for k in range(NBUF):
        fire(jnp.int32(k))

    def body(s, carry):
        @pl.when(s >= WD)
        def _():
            wait_write(s - WD)
            fire(s - WD + NBUF)

        wait_gather(s)
        start_write(s)
        return carry

    lax.fori_loop(0, n_w, body, 0)

    for k in range(WD):
        wait_write(n_w - WD + k)


_mesh = plsc.VectorSubcoreMesh(core_axis_name="c", subcore_axis_name="s")

_gather = functools.partial(
    pl.kernel,
    mesh=_mesh,
    out_type=jax.ShapeDtypeStruct((NUM_NODES, H_DIM), jnp.float32),
    scratch_types=[
        pltpu.VMEM((SLOTS * CHUNK,), jnp.int32),
        pltpu.VMEM((NBUF, CHUNK, H_DIM), jnp.float32),
        pltpu.SemaphoreType.DMA((NBUF,)),
        pltpu.SemaphoreType.DMA((NBUF,)),
    ],
)(_gather_body)


@jax.jit
def kernel(g, h, r, norm, W):
    idx = h.reshape(-1).astype(jnp.int32)
    return _gather(idx, W)
